# Initial kernel scaffold; baseline (speedup 1.0000x reference)
import jax
import jax.numpy as jnp
from jax.experimental import pallas as pl


def kernel(p_id, c_id, starts, paths, ends, masks, target_c, result, c_embed, cur_result, params):
    raise NotImplementedError("write your pallas kernel here")


# speedup vs baseline: 1.2902x; 1.2902x over previous
import functools

import jax
import jax.numpy as jnp
from jax import lax
from jax.experimental import pallas as pl
from jax.experimental.pallas import tpu as pltpu
from jax.experimental.pallas import tpu_sc as plsc

B, S, NP, CV, NC, D, H, DH = 2, 2048, 32, 64, 125, 192, 8, 24
NB, CHUNK = 32, 64
BS = B * S
BSN = BS * NP


_NW = 32
_ROWS_PER_W = BSN // _NW
_GCHUNK = 1024


def _sc_gather_body(node_hbm, path_hbm, starts_hbm, paths_hbm, ends_hbm,
                    s_out, p_out, e_out, idx_v, rows_v, sem):
    cid = lax.axis_index("c")
    sid = lax.axis_index("s")
    wid = sid * 2 + cid
    for table, idx_hbm, out in ((node_hbm, starts_hbm, s_out),
                                (path_hbm, paths_hbm, p_out),
                                (node_hbm, ends_hbm, e_out)):
        for ch in range(_ROWS_PER_W // _GCHUNK):
            base = wid * _ROWS_PER_W + ch * _GCHUNK
            pltpu.sync_copy(idx_hbm.at[pl.ds(base, _GCHUNK)], idx_v)
            pltpu.async_copy(table.at[idx_v], rows_v, sem).wait()
            pltpu.sync_copy(rows_v, out.at[pl.ds(base, _GCHUNK)])


def _sc_gather(node_emb, path_emb, starts_f, paths_f, ends_f):
    mesh = plsc.VectorSubcoreMesh(core_axis_name="c", subcore_axis_name="s")
    out_t = jax.ShapeDtypeStruct((BSN, CV), jnp.float32)
    kern = functools.partial(
        pl.kernel,
        out_type=[out_t, out_t, out_t],
        mesh=mesh,
        scratch_types=[
            pltpu.VMEM((_GCHUNK,), jnp.int32),
            pltpu.VMEM((_GCHUNK, CV), jnp.float32),
            pltpu.SemaphoreType.DMA,
        ],
        compiler_params=pltpu.CompilerParams(use_tc_tiling_on_sc=False),
    )(_sc_gather_body)
    return kern(node_emb, path_emb, starts_f, paths_f, ends_f)


_POOL_R = 256


def _pool_body(s_ref, p_ref, e_ref, wcv_ref, a_ref, out_ref):
    w1 = wcv_ref[0:CV, :]
    w2 = wcv_ref[CV:2 * CV, :]
    w3 = wcv_ref[2 * CV:3 * CV, :]
    a_col = a_ref[...]
    acc = jnp.zeros((_POOL_R, CV), jnp.float32)
    for n in range(NP):
        ctx = (jnp.dot(s_ref[n], w1, preferred_element_type=jnp.float32)
               + jnp.dot(p_ref[n], w2, preferred_element_type=jnp.float32)
               + jnp.dot(e_ref[n], w3, preferred_element_type=jnp.float32))
        cad = jnp.tanh(ctx)
        aw = jnp.dot(cad, a_col, preferred_element_type=jnp.float32)
        acc = acc + cad * aw
    out_ref[...] = acc


def _pool(s_e, p_e, e_e, wcv, a_col):
    grid = (BS // _POOL_R,)
    in_spec = pl.BlockSpec((NP, _POOL_R, CV), lambda i: (0, i, 0))
    return pl.pallas_call(
        _pool_body,
        grid=grid,
        in_specs=[in_spec, in_spec, in_spec,
                  pl.BlockSpec((3 * CV, CV), lambda i: (0, 0)),
                  pl.BlockSpec((CV, 1), lambda i: (0, 0))],
        out_specs=pl.BlockSpec((_POOL_R, CV), lambda i: (i, 0)),
        out_shape=jax.ShapeDtypeStruct((BS, CV), jnp.float32),
    )(s_e, p_e, e_e, wcv, a_col)


def _layer_norm(x, s, b):
    m = jnp.mean(x, axis=-1, keepdims=True)
    v = jnp.var(x, axis=-1, keepdims=True)
    return (x - m) / jnp.sqrt(v + 1e-5) * s + b


def _lsh_attention(x, p):
    b, s, d = x.shape
    qk = (x @ p["Wqk"]).reshape(b, s, H, DH).transpose(0, 2, 1, 3)
    v = (x @ p["Wv"]).reshape(b, s, H, DH).transpose(0, 2, 1, 3)
    rotated = jnp.einsum("bhsd,dr->bhsr", qk, p["rot"])
    buckets = jnp.argmax(jnp.concatenate([rotated, -rotated], axis=-1), axis=-1)
    pos = jnp.arange(s)
    order = jnp.argsort(buckets * s + pos, axis=-1)
    sqk = jnp.take_along_axis(qk, order[..., None], axis=2)
    sv = jnp.take_along_axis(v, order[..., None], axis=2)
    spos = jnp.take_along_axis(jnp.broadcast_to(pos, (b, H, s)), order, axis=2)
    nch = s // CHUNK
    q_c = sqk.reshape(b, H, nch, CHUNK, DH)
    k_n = sqk / (jnp.linalg.norm(sqk, axis=-1, keepdims=True) + 1e-6)
    k_c = k_n.reshape(b, H, nch, CHUNK, DH)
    v_c = sv.reshape(b, H, nch, CHUNK, DH)
    p_c = spos.reshape(b, H, nch, CHUNK)
    k_cat = jnp.concatenate([jnp.roll(k_c, 1, axis=2), k_c], axis=3)
    v_cat = jnp.concatenate([jnp.roll(v_c, 1, axis=2), v_c], axis=3)
    p_cat = jnp.concatenate([jnp.roll(p_c, 1, axis=2), p_c], axis=3)
    scores = jnp.einsum("bhncd,bhnkd->bhnck", q_c, k_cat) / jnp.sqrt(float(DH))
    qpos = p_c[..., :, None]
    kpos = p_cat[..., None, :]
    scores = jnp.where(kpos > qpos, -1e9, scores)
    scores = jnp.where(kpos == qpos, -1e5, scores)
    attn = jax.nn.softmax(scores, axis=-1)
    out = jnp.einsum("bhnck,bhnkd->bhncd", attn, v_cat).reshape(b, H, s, DH)
    inv = jnp.argsort(order, axis=-1)
    out = jnp.take_along_axis(out, inv[..., None], axis=2)
    out = out.transpose(0, 2, 1, 3).reshape(b, s, d)
    return out @ p["Wo"]


def _reformer(x, layers):
    for p in layers:
        x = x + _lsh_attention(_layer_norm(x, p["ln1_s"], p["ln1_b"]), p)
        h1 = jax.nn.gelu(_layer_norm(x, p["ln2_s"], p["ln2_b"]) @ p["W1"] + p["b1"])
        x = x + h1 @ p["W2"] + p["b2"]
    return x


def kernel(p_id, c_id, starts, paths, ends, masks, target_c, result, c_embed,
           cur_result, params):
    starts_f = starts.astype(jnp.int32).transpose(2, 0, 1).reshape(-1)
    paths_f = paths.astype(jnp.int32).transpose(2, 0, 1).reshape(-1)
    ends_f = ends.astype(jnp.int32).transpose(2, 0, 1).reshape(-1)

    s_e, p_e, e_e = _sc_gather(params["node_emb"], params["path_emb"],
                               starts_f, paths_f, ends_f)
    code_vectors = _pool(s_e.reshape(NP, BS, CV), p_e.reshape(NP, BS, CV),
                         e_e.reshape(NP, BS, CV), params["Wcv"],
                         params["a"].reshape(CV, 1)).reshape(B, S, CV)

    x = jnp.concatenate([c_embed, code_vectors, cur_result], axis=2)
    out = _reformer(x, params["layers"])
    pred = out.reshape(B * S, D) @ params["Wp"] + params["bp"]
    pred1d = jnp.sum(pred * target_c.reshape(B * S, NC + 1), axis=1)
    ncs = jnp.sum(target_c, axis=2).reshape(-1)
    m = (ncs > 0).astype(jnp.float32)
    fp = pred1d / jnp.where(m > 0, ncs, 1.0)
    ft = result[:, 0]
    per = jnp.maximum(fp, 0.0) - fp * ft + jnp.log1p(jnp.exp(-jnp.abs(fp)))
    loss = jnp.sum(per * m) / jnp.sum(m)
    return (loss, jax.nn.sigmoid(fp), ft)


# speedup vs baseline: 3.7813x; 2.9308x over previous
import functools

import jax
import jax.numpy as jnp
from jax import lax
from jax.experimental import pallas as pl
from jax.experimental.pallas import tpu as pltpu
from jax.experimental.pallas import tpu_sc as plsc

B, S, NP, CV, NC, D, H, DH = 2, 2048, 32, 64, 125, 192, 8, 24
NB, CHUNK = 32, 64
BS = B * S
BSN = BS * NP
BHS = B * H * S
NCH = S // CHUNK
PK = 64
AK = 32


_NW = 32
_ROWS_PER_W = BSN // _NW
_GCHUNK = 1024


def _sc_gather_body(node_hbm, path_hbm, starts_hbm, paths_hbm, ends_hbm,
                    s_out, p_out, e_out, idx_v, rows_v, sem):
    wid = lax.axis_index("s") * 2 + lax.axis_index("c")
    for table, idx_hbm, out in ((node_hbm, starts_hbm, s_out),
                                (path_hbm, paths_hbm, p_out),
                                (node_hbm, ends_hbm, e_out)):
        for ch in range(_ROWS_PER_W // _GCHUNK):
            base = wid * _ROWS_PER_W + ch * _GCHUNK
            pltpu.sync_copy(idx_hbm.at[pl.ds(base, _GCHUNK)], idx_v)
            pltpu.async_copy(table.at[idx_v], rows_v, sem).wait()
            pltpu.sync_copy(rows_v, out.at[pl.ds(base, _GCHUNK)])


def _sc_gather(node_emb, path_emb, starts_f, paths_f, ends_f):
    mesh = plsc.VectorSubcoreMesh(core_axis_name="c", subcore_axis_name="s")
    out_t = jax.ShapeDtypeStruct((BSN, CV), jnp.float32)
    kern = functools.partial(
        pl.kernel,
        out_type=[out_t, out_t, out_t],
        mesh=mesh,
        scratch_types=[
            pltpu.VMEM((_GCHUNK,), jnp.int32),
            pltpu.VMEM((_GCHUNK, CV), jnp.float32),
            pltpu.SemaphoreType.DMA,
        ],
        compiler_params=pltpu.CompilerParams(use_tc_tiling_on_sc=False),
    )(_sc_gather_body)
    return kern(node_emb, path_emb, starts_f, paths_f, ends_f)


_PCH = BHS // _NW


def _sc_scatter_body(rows_hbm, rank_hbm, out_hbm, idx_v, rows_v, sem):
    wid = lax.axis_index("s") * 2 + lax.axis_index("c")
    base = wid * _PCH
    pltpu.sync_copy(rank_hbm.at[pl.ds(base, _PCH)], idx_v)
    pltpu.sync_copy(rows_hbm.at[pl.ds(base, _PCH)], rows_v)
    pltpu.async_copy(rows_v, out_hbm.at[idx_v], sem).wait()


def _sc_scatter_rows(rows, rank):
    mesh = plsc.VectorSubcoreMesh(core_axis_name="c", subcore_axis_name="s")
    kern = functools.partial(
        pl.kernel,
        out_type=jax.ShapeDtypeStruct((BHS, PK), jnp.float32),
        mesh=mesh,
        scratch_types=[
            pltpu.VMEM((_PCH,), jnp.int32),
            pltpu.VMEM((_PCH, PK), jnp.float32),
            pltpu.SemaphoreType.DMA,
        ],
        compiler_params=pltpu.CompilerParams(use_tc_tiling_on_sc=False),
    )(_sc_scatter_body)
    return kern(rows, rank)


def _sc_unpermute_body(rows_hbm, rank_hbm, out_hbm, idx_v, rows_v, sem):
    wid = lax.axis_index("s") * 2 + lax.axis_index("c")
    base = wid * _PCH
    pltpu.sync_copy(rank_hbm.at[pl.ds(base, _PCH)], idx_v)
    pltpu.async_copy(rows_hbm.at[idx_v], rows_v, sem).wait()
    pltpu.sync_copy(rows_v, out_hbm.at[pl.ds(base, _PCH)])


def _sc_unpermute_rows(rows_sorted, rank):
    mesh = plsc.VectorSubcoreMesh(core_axis_name="c", subcore_axis_name="s")
    kern = functools.partial(
        pl.kernel,
        out_type=jax.ShapeDtypeStruct((BHS, AK), jnp.float32),
        mesh=mesh,
        scratch_types=[
            pltpu.VMEM((_PCH,), jnp.int32),
            pltpu.VMEM((_PCH, AK), jnp.float32),
            pltpu.SemaphoreType.DMA,
        ],
        compiler_params=pltpu.CompilerParams(use_tc_tiling_on_sc=False),
    )(_sc_unpermute_body)
    return kern(rows_sorted, rank)


_POOL_R = 256


def _pool_body(s_ref, p_ref, e_ref, wcv_ref, a_ref, out_ref):
    w1 = wcv_ref[0:CV, :]
    w2 = wcv_ref[CV:2 * CV, :]
    w3 = wcv_ref[2 * CV:3 * CV, :]
    a_col = a_ref[...]
    acc = jnp.zeros((_POOL_R, CV), jnp.float32)
    for n in range(NP):
        ctx = (jnp.dot(s_ref[n], w1, preferred_element_type=jnp.float32)
               + jnp.dot(p_ref[n], w2, preferred_element_type=jnp.float32)
               + jnp.dot(e_ref[n], w3, preferred_element_type=jnp.float32))
        cad = jnp.tanh(ctx)
        aw = jnp.dot(cad, a_col, preferred_element_type=jnp.float32)
        acc = acc + cad * aw
    out_ref[...] = acc


def _pool(s_e, p_e, e_e, wcv, a_col):
    in_spec = pl.BlockSpec((NP, _POOL_R, CV), lambda i: (0, i, 0))
    return pl.pallas_call(
        _pool_body,
        grid=(BS // _POOL_R,),
        in_specs=[in_spec, in_spec, in_spec,
                  pl.BlockSpec((3 * CV, CV), lambda i: (0, 0)),
                  pl.BlockSpec((CV, 1), lambda i: (0, 0))],
        out_specs=pl.BlockSpec((_POOL_R, CV), lambda i: (i, 0)),
        out_shape=jax.ShapeDtypeStruct((BS, CV), jnp.float32),
    )(s_e, p_e, e_e, wcv, a_col)


_R = 512
_NRB = S // _R


def _ln(x, s_row, b_row):
    m = jnp.mean(x, axis=-1, keepdims=True)
    v = jnp.mean((x - m) ** 2, axis=-1, keepdims=True)
    return (x - m) / jnp.sqrt(v + 1e-5) * s_row + b_row


def _emit_qkv(x, wqk_ref, wv_ref, rot_ref, ln_s_ref, ln_b_ref,
              qkv_ref, rot_out_ref, rblk):
    h_ln = _ln(x, ln_s_ref[...], ln_b_ref[...])
    qk = jnp.dot(h_ln, wqk_ref[...], preferred_element_type=jnp.float32)
    v = jnp.dot(h_ln, wv_ref[...], preferred_element_type=jnp.float32)
    pos = (lax.broadcasted_iota(jnp.int32, (_R, 1), 0)
           + _R * rblk).astype(jnp.float32)
    for h in range(H):
        qkh = qk[:, h * DH:(h + 1) * DH]
        qkv_ref[0, h, :, 0:DH] = qkh
        qkv_ref[0, h, :, DH:2 * DH] = v[:, h * DH:(h + 1) * DH]
        qkv_ref[0, h, :, 2 * DH:2 * DH + 1] = pos
        qkv_ref[0, h, :, 2 * DH + 1:PK] = jnp.zeros((_R, PK - 2 * DH - 1),
                                                    jnp.float32)
        rot_out_ref[0, h] = jnp.dot(qkh, rot_ref[...],
                                    preferred_element_type=jnp.float32)


def _qkv_body(x_ref, wqk_ref, wv_ref, rot_ref, ln_s_ref, ln_b_ref,
              qkv_ref, rot_out_ref):
    _emit_qkv(x_ref[0], wqk_ref, wv_ref, rot_ref, ln_s_ref, ln_b_ref,
              qkv_ref, rot_out_ref, pl.program_id(1))


def _qkv(x, p):
    return pl.pallas_call(
        _qkv_body,
        grid=(B, _NRB),
        in_specs=[
            pl.BlockSpec((1, _R, D), lambda b, r: (b, r, 0)),
            pl.BlockSpec((D, D), lambda b, r: (0, 0)),
            pl.BlockSpec((D, D), lambda b, r: (0, 0)),
            pl.BlockSpec((DH, NB // 2), lambda b, r: (0, 0)),
            pl.BlockSpec((1, D), lambda b, r: (0, 0)),
            pl.BlockSpec((1, D), lambda b, r: (0, 0)),
        ],
        out_specs=[
            pl.BlockSpec((1, H, _R, PK), lambda b, r: (b, 0, r, 0)),
            pl.BlockSpec((1, H, _R, NB // 2), lambda b, r: (b, 0, r, 0)),
        ],
        out_shape=[
            jax.ShapeDtypeStruct((B, H, S, PK), jnp.float32),
            jax.ShapeDtypeStruct((B, H, S, NB // 2), jnp.float32),
        ],
    )(x, p["Wqk"], p["Wv"], p["rot"], p["ln1_s"].reshape(1, D),
      p["ln1_b"].reshape(1, D))


def _rank_body(rot_ref, rank_ref, tri_ref):
    @pl.when(jnp.logical_and(pl.program_id(0) == 0, pl.program_id(1) == 0))
    def _():
        row = lax.broadcasted_iota(jnp.int32, (S, S), 0)
        col = lax.broadcasted_iota(jnp.int32, (S, S), 1)
        tri_ref[...] = (col <= row).astype(jnp.bfloat16)

    r = rot_ref[0, 0]
    a = jnp.concatenate([r, -r], axis=1)
    mx = jnp.max(a, axis=1, keepdims=True)
    lanes = lax.broadcasted_iota(jnp.int32, (S, NB), 1)
    bucket = jnp.min(jnp.where(a == mx, lanes, NB), axis=1, keepdims=True)
    onehot = (lanes == bucket).astype(jnp.float32)
    csum = jnp.dot(tri_ref[...], onehot.astype(jnp.bfloat16),
                   preferred_element_type=jnp.float32)
    cnt = jnp.sum(csum * onehot, axis=1, keepdims=True)
    totals = csum[S - 1:S, :]
    brow = lax.broadcasted_iota(jnp.int32, (NB, NB), 0)
    bcol = lax.broadcasted_iota(jnp.int32, (NB, NB), 1)
    strict = (brow < bcol).astype(jnp.float32)
    offs = jnp.dot(totals, strict, preferred_element_type=jnp.float32)
    off_of = jnp.sum(onehot * offs, axis=1, keepdims=True)
    base = pl.program_id(0) * H + pl.program_id(1)
    rank = off_of + cnt - 1.0 + jnp.float32(S) * base.astype(jnp.float32)
    rank_ref[0, 0] = rank.astype(jnp.int32)


def _rank(rot):
    return pl.pallas_call(
        _rank_body,
        grid=(B, H),
        in_specs=[pl.BlockSpec((1, 1, S, NB // 2), lambda b, h: (b, h, 0, 0))],
        out_specs=pl.BlockSpec((1, 1, S, 1), lambda b, h: (b, h, 0, 0)),
        out_shape=jax.ShapeDtypeStruct((B, H, S, 1), jnp.int32),
        scratch_shapes=[pltpu.VMEM((S, S), jnp.bfloat16)],
    )(rot)


def _attn_body(data_ref, out_ref):
    data = data_ref[0, 0]
    qk_s = data[:, 0:DH]
    v_s = data[:, DH:2 * DH]
    pos_col = data[:, 2 * DH:2 * DH + 1]
    pos_row = jnp.transpose(pos_col)
    norm = jnp.sqrt(jnp.sum(qk_s * qk_s, axis=1, keepdims=True))
    k_n = qk_s / (norm + 1e-6)
    scale = 1.0 / jnp.sqrt(jnp.float32(DH))
    for n in range(NCH):
        pv = ((n - 1) % NCH) * CHUNK
        cu = n * CHUNK
        q = qk_s[cu:cu + CHUNK, :]
        kw = jnp.concatenate([k_n[pv:pv + CHUNK, :],
                              k_n[cu:cu + CHUNK, :]], axis=0)
        vw = jnp.concatenate([v_s[pv:pv + CHUNK, :],
                              v_s[cu:cu + CHUNK, :]], axis=0)
        kp = jnp.concatenate([pos_row[:, pv:pv + CHUNK],
                              pos_row[:, cu:cu + CHUNK]], axis=1)
        qp = pos_col[cu:cu + CHUNK, :]
        scores = lax.dot_general(q, kw, (((1,), (1,)), ((), ())),
                                 preferred_element_type=jnp.float32) * scale
        scores = jnp.where(kp > qp, -1e9, scores)
        scores = jnp.where(kp == qp, -1e5, scores)
        mx = jnp.max(scores, axis=1, keepdims=True)
        p = jnp.exp(scores - mx)
        attn = p / jnp.sum(p, axis=1, keepdims=True)
        out = jnp.dot(attn, vw, preferred_element_type=jnp.float32)
        out_ref[0, 0, pl.ds(cu, CHUNK), 0:DH] = out
        out_ref[0, 0, pl.ds(cu, CHUNK), DH:AK] = jnp.zeros((CHUNK, AK - DH),
                                                           jnp.float32)


def _attn(sorted_qkv):
    return pl.pallas_call(
        _attn_body,
        grid=(B, H),
        in_specs=[pl.BlockSpec((1, 1, S, PK), lambda b, h: (b, h, 0, 0))],
        out_specs=pl.BlockSpec((1, 1, S, AK), lambda b, h: (b, h, 0, 0)),
        out_shape=jax.ShapeDtypeStruct((B, H, S, AK), jnp.float32),
    )(sorted_qkv)


def _mix_core(x, att_ref, wo_ref, ln2_s_ref, ln2_b_ref, w1_ref, b1_ref,
              w2_ref, b2_ref):
    att = jnp.concatenate([att_ref[0, h, :, 0:DH] for h in range(H)], axis=1)
    y = jnp.dot(att, wo_ref[...], preferred_element_type=jnp.float32)
    x1 = x + y
    h_ln = _ln(x1, ln2_s_ref[...], ln2_b_ref[...])
    h1 = jax.nn.gelu(jnp.dot(h_ln, w1_ref[...],
                             preferred_element_type=jnp.float32) + b1_ref[...])
    return x1 + jnp.dot(h1, w2_ref[...],
                        preferred_element_type=jnp.float32) + b2_ref[...]


def _mix_qkv_body(x_ref, att_ref, wo_ref, ln2_s_ref, ln2_b_ref, w1_ref,
                  b1_ref, w2_ref, b2_ref, wqk_ref, wv_ref, rot_ref,
                  ln1_s_ref, ln1_b_ref, x2_ref, qkv_ref, rot_out_ref):
    x2 = _mix_core(x_ref[0], att_ref, wo_ref, ln2_s_ref, ln2_b_ref,
                   w1_ref, b1_ref, w2_ref, b2_ref)
    x2_ref[0] = x2
    _emit_qkv(x2, wqk_ref, wv_ref, rot_ref, ln1_s_ref, ln1_b_ref,
              qkv_ref, rot_out_ref, pl.program_id(1))


def _mix_qkv(x, att, p, p_next):
    row = lambda k: p[k].reshape(1, D)
    return pl.pallas_call(
        _mix_qkv_body,
        grid=(B, _NRB),
        in_specs=[
            pl.BlockSpec((1, _R, D), lambda b, r: (b, r, 0)),
            pl.BlockSpec((1, H, _R, AK), lambda b, r: (b, 0, r, 0)),
            pl.BlockSpec((D, D), lambda b, r: (0, 0)),
            pl.BlockSpec((1, D), lambda b, r: (0, 0)),
            pl.BlockSpec((1, D), lambda b, r: (0, 0)),
            pl.BlockSpec((D, 4 * D), lambda b, r: (0, 0)),
            pl.BlockSpec((1, 4 * D), lambda b, r: (0, 0)),
            pl.BlockSpec((4 * D, D), lambda b, r: (0, 0)),
            pl.BlockSpec((1, D), lambda b, r: (0, 0)),
            pl.BlockSpec((D, D), lambda b, r: (0, 0)),
            pl.BlockSpec((D, D), lambda b, r: (0, 0)),
            pl.BlockSpec((DH, NB // 2), lambda b, r: (0, 0)),
            pl.BlockSpec((1, D), lambda b, r: (0, 0)),
            pl.BlockSpec((1, D), lambda b, r: (0, 0)),
        ],
        out_specs=[
            pl.BlockSpec((1, _R, D), lambda b, r: (b, r, 0)),
            pl.BlockSpec((1, H, _R, PK), lambda b, r: (b, 0, r, 0)),
            pl.BlockSpec((1, H, _R, NB // 2), lambda b, r: (b, 0, r, 0)),
        ],
        out_shape=[
            jax.ShapeDtypeStruct((B, S, D), jnp.float32),
            jax.ShapeDtypeStruct((B, H, S, PK), jnp.float32),
            jax.ShapeDtypeStruct((B, H, S, NB // 2), jnp.float32),
        ],
    )(x, att, p["Wo"], row("ln2_s"), row("ln2_b"), p["W1"],
      p["b1"].reshape(1, 4 * D), p["W2"], p["b2"].reshape(1, D),
      p_next["Wqk"], p_next["Wv"], p_next["rot"],
      p_next["ln1_s"].reshape(1, D), p_next["ln1_b"].reshape(1, D))


def _mix_head_body(x_ref, att_ref, wo_ref, ln2_s_ref, ln2_b_ref, w1_ref,
                   b1_ref, w2_ref, b2_ref, wp_ref, bp_ref, tc_ref, ft_ref,
                   fp_ref, parts_ref):
    x2 = _mix_core(x_ref[0], att_ref, wo_ref, ln2_s_ref, ln2_b_ref,
                   w1_ref, b1_ref, w2_ref, b2_ref)
    pred = jnp.dot(x2, wp_ref[...],
                   preferred_element_type=jnp.float32) + bp_ref[...]
    tc = tc_ref[0]
    pred1d = jnp.sum(pred * tc, axis=1, keepdims=True)
    ncs = jnp.sum(tc, axis=1, keepdims=True)
    mk = (ncs > 0).astype(jnp.float32)
    fp = pred1d / jnp.where(ncs > 0, ncs, 1.0)
    ft = ft_ref[0]
    per = (jnp.maximum(fp, 0.0) - fp * ft
           + jnp.log1p(jnp.exp(-jnp.abs(fp))))
    fp_ref[0] = 1.0 / (1.0 + jnp.exp(-fp))
    parts_ref[0, :, 0:1] = per * mk
    parts_ref[0, :, 1:2] = mk


def _mix_head(x, att, p, wp, bp, target_c, ft):
    row = lambda k: p[k].reshape(1, D)
    return pl.pallas_call(
        _mix_head_body,
        grid=(B, _NRB),
        in_specs=[
            pl.BlockSpec((1, _R, D), lambda b, r: (b, r, 0)),
            pl.BlockSpec((1, H, _R, AK), lambda b, r: (b, 0, r, 0)),
            pl.BlockSpec((D, D), lambda b, r: (0, 0)),
            pl.BlockSpec((1, D), lambda b, r: (0, 0)),
            pl.BlockSpec((1, D), lambda b, r: (0, 0)),
            pl.BlockSpec((D, 4 * D), lambda b, r: (0, 0)),
            pl.BlockSpec((1, 4 * D), lambda b, r: (0, 0)),
            pl.BlockSpec((4 * D, D), lambda b, r: (0, 0)),
            pl.BlockSpec((1, D), lambda b, r: (0, 0)),
            pl.BlockSpec((D, NC + 1), lambda b, r: (0, 0)),
            pl.BlockSpec((1, NC + 1), lambda b, r: (0, 0)),
            pl.BlockSpec((1, _R, NC + 1), lambda b, r: (b, r, 0)),
            pl.BlockSpec((1, _R, 1), lambda b, r: (b, r, 0)),
        ],
        out_specs=[
            pl.BlockSpec((1, _R, 1), lambda b, r: (b, r, 0)),
            pl.BlockSpec((1, _R, 2), lambda b, r: (b, r, 0)),
        ],
        out_shape=[
            jax.ShapeDtypeStruct((B, S, 1), jnp.float32),
            jax.ShapeDtypeStruct((B, S, 2), jnp.float32),
        ],
    )(x, att, p["Wo"], row("ln2_s"), row("ln2_b"), p["W1"],
      p["b1"].reshape(1, 4 * D), p["W2"], p["b2"].reshape(1, D),
      wp, bp.reshape(1, NC + 1), target_c, ft)


def kernel(p_id, c_id, starts, paths, ends, masks, target_c, result, c_embed,
           cur_result, params):
    starts_f = starts.astype(jnp.int32).transpose(2, 0, 1).reshape(-1)
    paths_f = paths.astype(jnp.int32).transpose(2, 0, 1).reshape(-1)
    ends_f = ends.astype(jnp.int32).transpose(2, 0, 1).reshape(-1)

    s_e, p_e, e_e = _sc_gather(params["node_emb"], params["path_emb"],
                               starts_f, paths_f, ends_f)
    code_vectors = _pool(s_e.reshape(NP, BS, CV), p_e.reshape(NP, BS, CV),
                         e_e.reshape(NP, BS, CV), params["Wcv"],
                         params["a"].reshape(CV, 1)).reshape(B, S, CV)

    x = jnp.concatenate([c_embed, code_vectors, cur_result], axis=2)

    layers = params["layers"]
    qkv, rot = _qkv(x, layers[0])
    for li, p in enumerate(layers):
        rank = _rank(rot).reshape(BHS)
        sorted_qkv = _sc_scatter_rows(qkv.reshape(BHS, PK), rank)
        att_sorted = _attn(sorted_qkv.reshape(B, H, S, PK))
        att = _sc_unpermute_rows(att_sorted.reshape(BHS, AK), rank)
        att = att.reshape(B, H, S, AK)
        if li + 1 < len(layers):
            x, qkv, rot = _mix_qkv(x, att, p, layers[li + 1])
        else:
            fp_sig, parts = _mix_head(x, att, p, params["Wp"], params["bp"],
                                      target_c, result.reshape(B, S, 1))

    loss = jnp.sum(parts[..., 0]) / jnp.sum(parts[..., 1])
    return (loss, fp_sig.reshape(BS), result[:, 0])
